# Initial kernel scaffold; baseline (speedup 1.0000x reference)
import jax
import jax.numpy as jnp
from jax.experimental import pallas as pl


def kernel(q, k, v):
    raise NotImplementedError("write your pallas kernel here")


# speedup vs baseline: 1.1185x; 1.1185x over previous
import functools

import jax
import jax.numpy as jnp
import numpy as np
from jax.experimental import pallas as pl
from jax.experimental.pallas import tpu as pltpu

_BLOCK = 64
_TOPK = 8


def _router_body(q_ref, k_ref, lut_ref, *, M, L, D):
    qp = jnp.mean(q_ref[0].reshape(M, _BLOCK, D), axis=1)
    kp = jnp.mean(k_ref[0].reshape(M, _BLOCK, D), axis=1)
    s = jax.lax.dot_general(qp.astype(jnp.bfloat16), kp.astype(jnp.bfloat16),
                            (((1,), (1,)), ((), ())),
                            preferred_element_type=jnp.float32)
    cols = jax.lax.broadcasted_iota(jnp.int32, (M, M), 1)
    picks = []
    for _ in range(_TOPK):
        mx = jnp.max(s, axis=-1, keepdims=True)
        idx = jnp.min(jnp.where(s == mx, cols, M), axis=-1, keepdims=True)
        picks.append(idx)
        s = jnp.where(cols == idx, -jnp.inf, s)
    lut = jnp.concatenate(picks, axis=1)
    lut_ref[0] = lut


def _attn_body(lut_ref, q_ref, k_ref, v_ref, o_ref, *, M, scale):
    bh = pl.program_id(0)
    m = pl.program_id(1)
    base = (bh * M + m) * _TOPK
    q = q_ref[0].astype(jnp.bfloat16)
    parts = []
    for t in range(_TOPK):
        idx = lut_ref[base + t]
        kt = k_ref[0, pl.ds(idx * _BLOCK, _BLOCK), :].astype(jnp.bfloat16)
        parts.append(jax.lax.dot_general(q, kt, (((1,), (1,)), ((), ())),
                                         preferred_element_type=jnp.float32))
    s = jnp.concatenate(parts, axis=-1) * scale
    mx = jnp.max(s, axis=-1, keepdims=True)
    p = jnp.exp(s - mx)
    se = jnp.sum(p, axis=-1, keepdims=True)
    p = p / se
    acc = None
    for t in range(_TOPK):
        idx = lut_ref[base + t]
        vt = v_ref[0, pl.ds(idx * _BLOCK, _BLOCK), :].astype(jnp.bfloat16)
        pt = p[:, t * _BLOCK:(t + 1) * _BLOCK].astype(jnp.bfloat16)
        contrib = jax.lax.dot_general(pt, vt, (((1,), (0,)), ((), ())),
                                      preferred_element_type=jnp.float32)
        acc = contrib if acc is None else acc + contrib
    o_ref[0] = acc


def kernel(q, k, v):
    B, H, L, D = q.shape
    M = L // _BLOCK
    BH = B * H
    q3 = q.reshape(BH, L, D)
    k3 = k.reshape(BH, L, D)
    v3 = v.reshape(BH, L, D)

    lut = pl.pallas_call(
        functools.partial(_router_body, M=M, L=L, D=D),
        grid=(BH,),
        in_specs=[pl.BlockSpec((1, L, D), lambda i: (i, 0, 0)),
                  pl.BlockSpec((1, L, D), lambda i: (i, 0, 0))],
        out_specs=pl.BlockSpec((1, M, _TOPK), lambda i: (i, 0, 0)),
        out_shape=jax.ShapeDtypeStruct((BH, M, _TOPK), jnp.int32),
    )(q3, k3)

    lut_flat = lut.reshape(BH * M * _TOPK)

    o = pl.pallas_call(
        functools.partial(_attn_body, M=M, scale=1.0 / np.sqrt(D)),
        grid_spec=pltpu.PrefetchScalarGridSpec(
            num_scalar_prefetch=1,
            grid=(BH, M),
            in_specs=[
                pl.BlockSpec((1, _BLOCK, D), lambda bh, m, lut: (bh, m, 0)),
                pl.BlockSpec((1, L, D), lambda bh, m, lut: (bh, 0, 0)),
                pl.BlockSpec((1, L, D), lambda bh, m, lut: (bh, 0, 0)),
            ],
            out_specs=pl.BlockSpec((1, _BLOCK, D), lambda bh, m, lut: (bh, m, 0)),
        ),
        out_shape=jax.ShapeDtypeStruct((BH, L, D), jnp.float32),
    )(lut_flat, q3, k3, v3)

    return o.reshape(B, H, L, D)


# speedup vs baseline: 2.3438x; 2.0956x over previous
import functools

import jax
import jax.numpy as jnp
import numpy as np
from jax.experimental import pallas as pl
from jax.experimental.pallas import tpu as pltpu

_BLOCK = 64
_TOPK = 8


def _router_body(q_ref, k_ref, lut_ref, *, M, L, D):
    qp = jnp.mean(q_ref[0].reshape(M, _BLOCK, D), axis=1)
    kp = jnp.mean(k_ref[0].reshape(M, _BLOCK, D), axis=1)
    s = jax.lax.dot_general(qp.astype(jnp.bfloat16), kp.astype(jnp.bfloat16),
                            (((1,), (1,)), ((), ())),
                            preferred_element_type=jnp.float32)
    cols = jax.lax.broadcasted_iota(jnp.int32, (M, M), 1)
    picks = []
    for _ in range(_TOPK):
        mx = jnp.max(s, axis=-1, keepdims=True)
        idx = jnp.min(jnp.where(s == mx, cols, M), axis=-1, keepdims=True)
        picks.append(idx)
        s = jnp.where(cols == idx, -jnp.inf, s)
    lut = jnp.concatenate(picks, axis=1)
    lut_ref[0] = lut


def _attn_body(lut_ref, q_ref, k_ref, v_ref, o_ref, kb_ref, vb_ref, *, M, scale):
    bh = pl.program_id(0)
    kb_ref[...] = k_ref[0].astype(jnp.bfloat16)
    vb_ref[...] = v_ref[0].astype(jnp.bfloat16)
    for m in range(M):
        base = (bh * M + m) * _TOPK
        q = q_ref[0, m * _BLOCK:(m + 1) * _BLOCK, :].astype(jnp.bfloat16)
        kcat = jnp.concatenate(
            [kb_ref[pl.ds(lut_ref[base + t] * _BLOCK, _BLOCK), :]
             for t in range(_TOPK)], axis=0)
        s = jax.lax.dot_general(q, kcat, (((1,), (1,)), ((), ())),
                                preferred_element_type=jnp.float32) * scale
        mx = jnp.max(s, axis=-1, keepdims=True)
        p = jnp.exp(s - mx)
        se = jnp.sum(p, axis=-1, keepdims=True)
        pb = (p / se).astype(jnp.bfloat16)
        vcat = jnp.concatenate(
            [vb_ref[pl.ds(lut_ref[base + t] * _BLOCK, _BLOCK), :]
             for t in range(_TOPK)], axis=0)
        o_ref[0, m * _BLOCK:(m + 1) * _BLOCK, :] = jax.lax.dot_general(
            pb, vcat, (((1,), (0,)), ((), ())),
            preferred_element_type=jnp.float32)


def kernel(q, k, v):
    B, H, L, D = q.shape
    M = L // _BLOCK
    BH = B * H
    q3 = q.reshape(BH, L, D)
    k3 = k.reshape(BH, L, D)
    v3 = v.reshape(BH, L, D)

    lut = pl.pallas_call(
        functools.partial(_router_body, M=M, L=L, D=D),
        grid=(BH,),
        in_specs=[pl.BlockSpec((1, L, D), lambda i: (i, 0, 0)),
                  pl.BlockSpec((1, L, D), lambda i: (i, 0, 0))],
        out_specs=pl.BlockSpec((1, M, _TOPK), lambda i: (i, 0, 0)),
        out_shape=jax.ShapeDtypeStruct((BH, M, _TOPK), jnp.int32),
    )(q3, k3)

    lut_flat = lut.reshape(BH * M * _TOPK)

    o = pl.pallas_call(
        functools.partial(_attn_body, M=M, scale=1.0 / np.sqrt(D)),
        grid_spec=pltpu.PrefetchScalarGridSpec(
            num_scalar_prefetch=1,
            grid=(BH,),
            in_specs=[
                pl.BlockSpec((1, L, D), lambda bh, lut: (bh, 0, 0)),
                pl.BlockSpec((1, L, D), lambda bh, lut: (bh, 0, 0)),
                pl.BlockSpec((1, L, D), lambda bh, lut: (bh, 0, 0)),
            ],
            out_specs=pl.BlockSpec((1, L, D), lambda bh, lut: (bh, 0, 0)),
            scratch_shapes=[pltpu.VMEM((L, D), jnp.bfloat16),
                            pltpu.VMEM((L, D), jnp.bfloat16)],
        ),
        out_shape=jax.ShapeDtypeStruct((BH, L, D), jnp.float32),
    )(lut_flat, q3, k3, v3)

    return o.reshape(B, H, L, D)


# speedup vs baseline: 6.3960x; 2.7289x over previous
import functools

import jax
import jax.numpy as jnp
import numpy as np
from jax.experimental import pallas as pl
from jax.experimental.pallas import tpu as pltpu

_BLOCK = 64
_TOPK = 8


def _router_body(q_ref, k_ref, lut_ref, *, M, L, D):
    qp = jnp.mean(q_ref[0].reshape(M, _BLOCK, D), axis=1)
    kp = jnp.mean(k_ref[0].reshape(M, _BLOCK, D), axis=1)
    s = jax.lax.dot_general(qp.astype(jnp.bfloat16), kp.astype(jnp.bfloat16),
                            (((1,), (1,)), ((), ())),
                            preferred_element_type=jnp.float32)
    cols = jax.lax.broadcasted_iota(jnp.int32, (M, M), 1)
    picks = []
    for _ in range(_TOPK):
        mx = jnp.max(s, axis=-1, keepdims=True)
        idx = jnp.min(jnp.where(s == mx, cols, M), axis=-1, keepdims=True)
        picks.append(idx)
        s = jnp.where(cols == idx, -jnp.inf, s)
    lut = jnp.concatenate(picks, axis=1)
    lut_ref[0] = lut


def _attn_body(lut_ref, q_ref, k_ref, v_ref, o_ref, kb_ref, vb_ref,
               s_ref, p_ref, *, M, scale):
    bh = pl.program_id(0)
    kb_ref[...] = k_ref[0].astype(jnp.bfloat16)
    vb_ref[...] = v_ref[0].astype(jnp.bfloat16)
    for m in range(M):
        base = (bh * M + m) * _TOPK
        q = q_ref[0, m * _BLOCK:(m + 1) * _BLOCK, :].astype(jnp.bfloat16)
        kcat = jnp.concatenate(
            [kb_ref[pl.ds(lut_ref[base + t] * _BLOCK, _BLOCK), :]
             for t in range(_TOPK)], axis=0)
        s_ref[m * _BLOCK:(m + 1) * _BLOCK, :] = jax.lax.dot_general(
            q, kcat, (((1,), (1,)), ((), ())),
            preferred_element_type=jnp.float32) * scale
    for m in range(M):
        s = s_ref[m * _BLOCK:(m + 1) * _BLOCK, :]
        mx = jnp.max(s, axis=-1, keepdims=True)
        p = jnp.exp(s - mx)
        se = jnp.sum(p, axis=-1, keepdims=True)
        p_ref[m * _BLOCK:(m + 1) * _BLOCK, :] = (p / se).astype(jnp.bfloat16)
    for m in range(M):
        base = (bh * M + m) * _TOPK
        vcat = jnp.concatenate(
            [vb_ref[pl.ds(lut_ref[base + t] * _BLOCK, _BLOCK), :]
             for t in range(_TOPK)], axis=0)
        o_ref[0, m * _BLOCK:(m + 1) * _BLOCK, :] = jax.lax.dot_general(
            p_ref[m * _BLOCK:(m + 1) * _BLOCK, :], vcat,
            (((1,), (0,)), ((), ())), preferred_element_type=jnp.float32)


def kernel(q, k, v):
    B, H, L, D = q.shape
    M = L // _BLOCK
    BH = B * H
    q3 = q.reshape(BH, L, D)
    k3 = k.reshape(BH, L, D)
    v3 = v.reshape(BH, L, D)

    lut = pl.pallas_call(
        functools.partial(_router_body, M=M, L=L, D=D),
        grid=(BH,),
        in_specs=[pl.BlockSpec((1, L, D), lambda i: (i, 0, 0)),
                  pl.BlockSpec((1, L, D), lambda i: (i, 0, 0))],
        out_specs=pl.BlockSpec((1, M, _TOPK), lambda i: (i, 0, 0)),
        out_shape=jax.ShapeDtypeStruct((BH, M, _TOPK), jnp.int32),
    )(q3, k3)

    lut_flat = lut.reshape(BH * M * _TOPK)

    o = pl.pallas_call(
        functools.partial(_attn_body, M=M, scale=1.0 / np.sqrt(D)),
        grid_spec=pltpu.PrefetchScalarGridSpec(
            num_scalar_prefetch=1,
            grid=(BH,),
            in_specs=[
                pl.BlockSpec((1, L, D), lambda bh, lut: (bh, 0, 0)),
                pl.BlockSpec((1, L, D), lambda bh, lut: (bh, 0, 0)),
                pl.BlockSpec((1, L, D), lambda bh, lut: (bh, 0, 0)),
            ],
            out_specs=pl.BlockSpec((1, L, D), lambda bh, lut: (bh, 0, 0)),
            scratch_shapes=[pltpu.VMEM((L, D), jnp.bfloat16),
                            pltpu.VMEM((L, D), jnp.bfloat16),
                            pltpu.VMEM((L, _TOPK * _BLOCK), jnp.float32),
                            pltpu.VMEM((L, _TOPK * _BLOCK), jnp.bfloat16)],
        ),
        out_shape=jax.ShapeDtypeStruct((BH, L, D), jnp.float32),
    )(lut_flat, q3, k3, v3)

    return o.reshape(B, H, L, D)


# speedup vs baseline: 8.7469x; 1.3676x over previous
import functools

import jax
import jax.numpy as jnp
import numpy as np
from jax.experimental import pallas as pl
from jax.experimental.pallas import tpu as pltpu

_BLOCK = 64
_TOPK = 8


def _router_body(q_ref, k_ref, lut_ref, *, M, L, D, G):
    for g in range(G):
        qp = jnp.mean(q_ref[g].reshape(M, _BLOCK, D), axis=1)
        kp = jnp.mean(k_ref[g].reshape(M, _BLOCK, D), axis=1)
        s = jax.lax.dot_general(qp.astype(jnp.bfloat16), kp.astype(jnp.bfloat16),
                                (((1,), (1,)), ((), ())),
                                preferred_element_type=jnp.float32)
        cols = jax.lax.broadcasted_iota(jnp.int32, (M, M), 1)
        picks = []
        for _ in range(_TOPK):
            mx = jnp.max(s, axis=-1, keepdims=True)
            idx = jnp.min(jnp.where(s == mx, cols, M), axis=-1, keepdims=True)
            picks.append(idx)
            s = jnp.where(cols == idx, -jnp.inf, s)
        lut = jnp.concatenate(picks, axis=1)
        lut_ref[g] = lut


def _attn_body(lut_ref, q_ref, k_ref, v_ref, o_ref, kb_ref, vb_ref,
               s_ref, p_ref, *, M, scale):
    bh = pl.program_id(0)
    kb_ref[...] = k_ref[0].astype(jnp.bfloat16)
    vb_ref[...] = v_ref[0].astype(jnp.bfloat16)
    for m in range(M):
        base = (bh * M + m) * _TOPK
        q = q_ref[0, m * _BLOCK:(m + 1) * _BLOCK, :].astype(jnp.bfloat16)
        kcat = jnp.concatenate(
            [kb_ref[pl.ds(lut_ref[base + t] * _BLOCK, _BLOCK), :]
             for t in range(_TOPK)], axis=0)
        s_ref[m * _BLOCK:(m + 1) * _BLOCK, :] = jax.lax.dot_general(
            q, kcat, (((1,), (1,)), ((), ())),
            preferred_element_type=jnp.float32) * scale
    for m in range(M):
        s = s_ref[m * _BLOCK:(m + 1) * _BLOCK, :]
        mx = jnp.max(s, axis=-1, keepdims=True)
        p = jnp.exp(s - mx)
        se = jnp.sum(p, axis=-1, keepdims=True)
        p_ref[m * _BLOCK:(m + 1) * _BLOCK, :] = (p / se).astype(jnp.bfloat16)
    for m in range(M):
        base = (bh * M + m) * _TOPK
        vcat = jnp.concatenate(
            [vb_ref[pl.ds(lut_ref[base + t] * _BLOCK, _BLOCK), :]
             for t in range(_TOPK)], axis=0)
        o_ref[0, m * _BLOCK:(m + 1) * _BLOCK, :] = jax.lax.dot_general(
            p_ref[m * _BLOCK:(m + 1) * _BLOCK, :], vcat,
            (((1,), (0,)), ((), ())), preferred_element_type=jnp.float32)


def kernel(q, k, v):
    B, H, L, D = q.shape
    M = L // _BLOCK
    BH = B * H
    q3 = q.reshape(BH, L, D)
    k3 = k.reshape(BH, L, D)
    v3 = v.reshape(BH, L, D)

    G = 4
    lut = pl.pallas_call(
        functools.partial(_router_body, M=M, L=L, D=D, G=G),
        grid=(BH // G,),
        in_specs=[pl.BlockSpec((G, L, D), lambda i: (i, 0, 0)),
                  pl.BlockSpec((G, L, D), lambda i: (i, 0, 0))],
        out_specs=pl.BlockSpec((G, M, _TOPK), lambda i: (i, 0, 0)),
        out_shape=jax.ShapeDtypeStruct((BH, M, _TOPK), jnp.int32),
    )(q3, k3)

    lut_flat = lut.reshape(BH * M * _TOPK)

    o = pl.pallas_call(
        functools.partial(_attn_body, M=M, scale=1.0 / np.sqrt(D)),
        grid_spec=pltpu.PrefetchScalarGridSpec(
            num_scalar_prefetch=1,
            grid=(BH,),
            in_specs=[
                pl.BlockSpec((1, L, D), lambda bh, lut: (bh, 0, 0)),
                pl.BlockSpec((1, L, D), lambda bh, lut: (bh, 0, 0)),
                pl.BlockSpec((1, L, D), lambda bh, lut: (bh, 0, 0)),
            ],
            out_specs=pl.BlockSpec((1, L, D), lambda bh, lut: (bh, 0, 0)),
            scratch_shapes=[pltpu.VMEM((L, D), jnp.bfloat16),
                            pltpu.VMEM((L, D), jnp.bfloat16),
                            pltpu.VMEM((L, _TOPK * _BLOCK), jnp.float32),
                            pltpu.VMEM((L, _TOPK * _BLOCK), jnp.bfloat16)],
        ),
        out_shape=jax.ShapeDtypeStruct((BH, L, D), jnp.float32),
    )(lut_flat, q3, k3, v3)

    return o.reshape(B, H, L, D)
